# baseline (speedup 1.0000x reference)
import functools

import jax
import jax.numpy as jnp
from jax import lax
from jax.experimental import pallas as pl
from jax.experimental.pallas import tpu as pltpu
from jax.experimental.pallas import tpu_sc as plsc

IN_CHANNELS = 9
HIDDEN = 128
EMB = 200
HALF = HIDDEN // 2
L = 16
NUM_WORKERS = 32
C_ROWS = 32
XW = C_ROWS * IN_CHANNELS
XPAD = XW + L
OW = C_ROWS * HIDDEN


def _body(x_hbm, tab_hbm, out_hbm, tab_v, x_v, out_v, sem_x, sem_o):
    n_rows = x_hbm.shape[0] // IN_CHANNELS
    n_chunks = n_rows // C_ROWS
    q, r = n_chunks // NUM_WORKERS, n_chunks % NUM_WORKERS

    wid = lax.axis_index("s") * 2 + lax.axis_index("c")
    my_chunks = q + jnp.where(wid < r, 1, 0)
    c0 = wid * q + jnp.minimum(wid, r)

    def x_src(k):
        return x_hbm.at[pl.ds((c0 + k) * XW, XW)]

    def x_dst(buf):
        return x_v.at[pl.ds(buf * XPAD, XW)]

    def out_src(buf):
        return out_v.at[pl.ds(buf * OW, OW)]

    def out_dst(k):
        return out_hbm.at[pl.ds((c0 + k) * C_ROWS * HIDDEN, OW)]

    pltpu.sync_copy(tab_hbm, tab_v)

    @pl.when(my_chunks > 0)
    def _prefetch0():
        pltpu.async_copy(x_src(0), x_dst(0), sem_x)

    def chunk_body(k, carry):
        buf = k & 1
        pltpu.make_async_copy(x_src(k), x_dst(buf), sem_x).wait()

        @pl.when(k + 1 < my_chunks)
        def _prefetch_next():
            pltpu.async_copy(x_src(k + 1), x_dst(1 - buf), sem_x)

        @pl.when(k >= 2)
        def _drain_out():
            pltpu.make_async_copy(out_src(buf), out_dst(k), sem_o).wait()

        xbase = buf * XPAD
        obase = buf * OW

        def row_body(rr, carry2):
            xv = x_v[pl.ds(xbase + rr * IN_CHANNELS, L)]
            offs = [(xv[i] + i * EMB) * HIDDEN for i in range(IN_CHANNELS)]
            for jc in range(HIDDEN // (2 * L)):
                vals = [
                    tab_v[pl.ds(offs[i] + jc * 2 * L, 2 * L)]
                    for i in range(IN_CHANNELS)
                ]
                while len(vals) > 1:
                    nxt = [
                        vals[i] + vals[i + 1] for i in range(0, len(vals) - 1, 2)
                    ]
                    if len(vals) % 2:
                        nxt.append(vals[-1])
                    vals = nxt
                out_v[pl.ds(obase + rr * HIDDEN + jc * 2 * L, 2 * L)] = vals[0]
            return carry2

        lax.fori_loop(0, C_ROWS, row_body, 0)
        pltpu.async_copy(out_src(buf), out_dst(k), sem_o)
        return carry

    lax.fori_loop(0, my_chunks, chunk_body, 0)

    def drain_body(i, carry):
        pltpu.make_async_copy(out_src(0), out_dst(0), sem_o).wait()
        return carry

    lax.fori_loop(0, jnp.minimum(my_chunks, 2), drain_body, 0)


def kernel(x, tables):
    n = x.shape[0]
    x_flat = x.reshape(-1)
    packed = tables.reshape(-1).astype(jnp.bfloat16)

    mesh = plsc.VectorSubcoreMesh(core_axis_name="c", subcore_axis_name="s")
    run = functools.partial(
        pl.kernel,
        out_type=jax.ShapeDtypeStruct((n * HIDDEN,), jnp.bfloat16),
        mesh=mesh,
        scratch_types=[
            pltpu.VMEM((IN_CHANNELS * EMB * HIDDEN,), jnp.bfloat16),
            pltpu.VMEM((2 * XPAD,), jnp.int32),
            pltpu.VMEM((2 * OW,), jnp.bfloat16),
            pltpu.SemaphoreType.DMA,
            pltpu.SemaphoreType.DMA,
        ],
        compiler_params=pltpu.CompilerParams(use_tc_tiling_on_sc=False),
    )(_body)
    return run(x_flat, packed).reshape(n, HIDDEN).astype(jnp.float32)


# speedup vs baseline: 1.2904x; 1.2904x over previous
import functools

import jax
import jax.numpy as jnp
from jax import lax
from jax.experimental import pallas as pl
from jax.experimental.pallas import tpu as pltpu
from jax.experimental.pallas import tpu_sc as plsc

IN_CHANNELS = 9
HIDDEN = 128
EMB = 200
HALF = HIDDEN // 2
L = 16
NUM_WORKERS = 32
C_ROWS = 32
XW = C_ROWS * IN_CHANNELS
XPAD = XW + L


def _body(x_hbm, tab_hbm, out_hbm, tab_v, x_v, out_v, sem_x, sem_o):
    n_rows = x_hbm.shape[0] // IN_CHANNELS
    n_chunks = n_rows // C_ROWS
    q, r = n_chunks // NUM_WORKERS, n_chunks % NUM_WORKERS

    wid = lax.axis_index("s") * 2 + lax.axis_index("c")
    my_chunks = q + jnp.where(wid < r, 1, 0)
    c0 = wid * q + jnp.minimum(wid, r)

    def x_src(k):
        return x_hbm.at[pl.ds((c0 + k) * XW, XW)]

    def x_dst(buf):
        return x_v.at[pl.ds(buf * XPAD, XW)]

    def out_src(buf):
        return out_v.at[buf]

    def out_dst(k):
        return out_hbm.at[pl.ds((c0 + k) * C_ROWS, C_ROWS), :]

    pltpu.sync_copy(tab_hbm, tab_v)
    hi_mask = jnp.int32(-65536)

    @pl.when(my_chunks > 0)
    def _prefetch0():
        pltpu.async_copy(x_src(0), x_dst(0), sem_x)

    def chunk_body(k, carry):
        buf = k & 1
        pltpu.make_async_copy(x_src(k), x_dst(buf), sem_x).wait()

        @pl.when(k + 1 < my_chunks)
        def _prefetch_next():
            pltpu.async_copy(x_src(k + 1), x_dst(1 - buf), sem_x)

        @pl.when(k >= 2)
        def _drain_out():
            pltpu.make_async_copy(out_src(buf), out_dst(k), sem_o).wait()

        xbase = buf * XPAD

        def row_body(rr, carry2):
            xv = x_v[pl.ds(xbase + rr * IN_CHANNELS, L)]
            offs = [(xv[i] + i * EMB) * HALF for i in range(IN_CHANNELS)]
            for jc in range(HALF // L):
                gs = [
                    tab_v[pl.ds(offs[i] + jc * L, L)] for i in range(IN_CHANNELS)
                ]
                vals_a = [
                    lax.bitcast_convert_type(g << 16, jnp.float32) for g in gs
                ]
                vals_b = [
                    lax.bitcast_convert_type(g & hi_mask, jnp.float32) for g in gs
                ]
                for vals in (vals_a, vals_b):
                    while len(vals) > 1:
                        nxt = [
                            vals[i] + vals[i + 1] for i in range(0, len(vals) - 1, 2)
                        ]
                        if len(vals) % 2:
                            nxt.append(vals[-1])
                        vals[:] = nxt
                out_v[buf, rr, pl.ds(jc * L, L)] = vals_a[0]
                out_v[buf, rr, pl.ds(HALF + jc * L, L)] = vals_b[0]
            return carry2

        lax.fori_loop(0, C_ROWS, row_body, 0)
        pltpu.async_copy(out_src(buf), out_dst(k), sem_o)
        return carry

    lax.fori_loop(0, my_chunks, chunk_body, 0)

    def drain_body(i, carry):
        pltpu.make_async_copy(out_src(0), out_dst(0), sem_o).wait()
        return carry

    lax.fori_loop(0, jnp.minimum(my_chunks, 2), drain_body, 0)


def kernel(x, tables):
    n = x.shape[0]
    x_flat = x.reshape(-1)
    tb = tables.reshape(IN_CHANNELS * EMB, HIDDEN).astype(jnp.bfloat16)
    pairs = jnp.stack([tb[:, :HALF], tb[:, HALF:]], axis=-1)
    packed = jax.lax.bitcast_convert_type(pairs, jnp.int32).reshape(-1)

    mesh = plsc.VectorSubcoreMesh(core_axis_name="c", subcore_axis_name="s")
    run = functools.partial(
        pl.kernel,
        out_type=jax.ShapeDtypeStruct((n, HIDDEN), jnp.float32),
        mesh=mesh,
        scratch_types=[
            pltpu.VMEM((IN_CHANNELS * EMB * HALF,), jnp.int32),
            pltpu.VMEM((2 * XPAD,), jnp.int32),
            pltpu.VMEM((2, C_ROWS, HIDDEN), jnp.float32),
            pltpu.SemaphoreType.DMA,
            pltpu.SemaphoreType.DMA,
        ],
    )(_body)
    return run(x_flat, packed)
